# baseline (speedup 1.0000x reference)
import functools

import jax
import jax.numpy as jnp
from jax.experimental import pallas as pl

_NUM_EXPERTS = 64
_TOP_K = 8
_BLOCK_T = 256


def _router_block(x_ref, w_ref, b_ref, out_ref):
    x = x_ref[...]
    w = w_ref[...]
    b = b_ref[...]
    logits = jnp.dot(x, w, preferred_element_type=jnp.float32) + b[None, :]

    rows = logits.shape[0]
    e_iota = jax.lax.broadcasted_iota(jnp.int32, (rows, _NUM_EXPERTS), 1)

    remaining = logits
    mask = jnp.zeros((rows, _NUM_EXPERTS), dtype=jnp.bool_)
    row_max = jnp.max(logits, axis=1, keepdims=True)
    for _ in range(_TOP_K):
        m = jnp.max(remaining, axis=1, keepdims=True)
        cand = jnp.where(remaining == m, e_iota, _NUM_EXPERTS)
        sel = jnp.min(cand, axis=1, keepdims=True)
        sel_mask = e_iota == sel
        mask = mask | sel_mask
        remaining = jnp.where(sel_mask, -jnp.inf, remaining)

    expw = jnp.where(mask, jnp.exp(logits - row_max), 0.0)
    out_ref[...] = expw / jnp.sum(expw, axis=1, keepdims=True)


@functools.partial(jax.jit, static_argnames=())
def kernel(hidden_states, W, b):
    tokens, d_model = hidden_states.shape
    grid = (tokens // _BLOCK_T,)
    return pl.pallas_call(
        _router_block,
        grid=grid,
        in_specs=[
            pl.BlockSpec((_BLOCK_T, d_model), lambda i: (i, 0)),
            pl.BlockSpec((d_model, _NUM_EXPERTS), lambda i: (0, 0)),
            pl.BlockSpec((_NUM_EXPERTS,), lambda i: (0,)),
        ],
        out_specs=pl.BlockSpec((_BLOCK_T, _NUM_EXPERTS), lambda i: (i, 0)),
        out_shape=jax.ShapeDtypeStruct((tokens, _NUM_EXPERTS), jnp.float32),
    )(hidden_states, W, b)


# speedup vs baseline: 1.7308x; 1.7308x over previous
import functools

import jax
import jax.numpy as jnp
from jax.experimental import pallas as pl

_NUM_EXPERTS = 64
_TOP_K = 8
_BLOCK_T = 2048


def _router_block(x_ref, w_ref, b_ref, out_ref):
    x = x_ref[...]
    w = w_ref[...]
    b = b_ref[...]
    logits = jnp.dot(x, w, preferred_element_type=jnp.float32) + b[None, :]

    rows = logits.shape[0]
    e_iota = jax.lax.broadcasted_iota(jnp.int32, (rows, _NUM_EXPERTS), 1)

    remaining = logits
    mask = jnp.zeros((rows, _NUM_EXPERTS), dtype=jnp.bool_)
    row_max = jnp.max(logits, axis=1, keepdims=True)
    for _ in range(_TOP_K):
        m = jnp.max(remaining, axis=1, keepdims=True)
        cand = jnp.where(remaining == m, e_iota, _NUM_EXPERTS)
        sel = jnp.min(cand, axis=1, keepdims=True)
        sel_mask = e_iota == sel
        mask = mask | sel_mask
        remaining = jnp.where(sel_mask, -jnp.inf, remaining)

    expw = jnp.where(mask, jnp.exp(logits - row_max), 0.0)
    out_ref[...] = expw / jnp.sum(expw, axis=1, keepdims=True)


@functools.partial(jax.jit, static_argnames=())
def kernel(hidden_states, W, b):
    tokens, d_model = hidden_states.shape
    grid = (tokens // _BLOCK_T,)
    return pl.pallas_call(
        _router_block,
        grid=grid,
        in_specs=[
            pl.BlockSpec((_BLOCK_T, d_model), lambda i: (i, 0)),
            pl.BlockSpec((d_model, _NUM_EXPERTS), lambda i: (0, 0)),
            pl.BlockSpec((_NUM_EXPERTS,), lambda i: (0,)),
        ],
        out_specs=pl.BlockSpec((_BLOCK_T, _NUM_EXPERTS), lambda i: (i, 0)),
        out_shape=jax.ShapeDtypeStruct((tokens, _NUM_EXPERTS), jnp.float32),
    )(hidden_states, W, b)


# speedup vs baseline: 2.6228x; 1.5154x over previous
import functools

import jax
import jax.numpy as jnp
from jax.experimental import pallas as pl

_NUM_EXPERTS = 64
_TOP_K = 8
_BLOCK_T = 1024


def _router_block(x_ref, w_ref, b_ref, out_ref):
    x = x_ref[...]
    w = w_ref[...]
    b = b_ref[...]
    logits = jnp.dot(x, w, preferred_element_type=jnp.float32) + b[None, :]

    lt = logits.T
    rows = lt.shape[1]
    e_iota = jax.lax.broadcasted_iota(jnp.int32, (_NUM_EXPERTS, rows), 0)

    remaining = lt
    mask = jnp.zeros((_NUM_EXPERTS, rows), dtype=jnp.bool_)
    row_max = None
    for k in range(_TOP_K):
        m = jnp.max(remaining, axis=0, keepdims=True)
        if k == 0:
            row_max = m
        cand = jnp.where(remaining == m, e_iota, _NUM_EXPERTS)
        sel = jnp.min(cand, axis=0, keepdims=True)
        sel_mask = e_iota == sel
        mask = mask | sel_mask
        remaining = jnp.where(sel_mask, -jnp.inf, remaining)

    expw = jnp.where(mask, jnp.exp(lt - row_max), 0.0)
    inv = 1.0 / jnp.sum(expw, axis=0, keepdims=True)
    out_ref[...] = (expw * inv).T


@functools.partial(jax.jit, static_argnames=())
def kernel(hidden_states, W, b):
    tokens, d_model = hidden_states.shape
    grid = (tokens // _BLOCK_T,)
    return pl.pallas_call(
        _router_block,
        grid=grid,
        in_specs=[
            pl.BlockSpec((_BLOCK_T, d_model), lambda i: (i, 0)),
            pl.BlockSpec((d_model, _NUM_EXPERTS), lambda i: (0, 0)),
            pl.BlockSpec((_NUM_EXPERTS,), lambda i: (0,)),
        ],
        out_specs=pl.BlockSpec((_BLOCK_T, _NUM_EXPERTS), lambda i: (i, 0)),
        out_shape=jax.ShapeDtypeStruct((tokens, _NUM_EXPERTS), jnp.float32),
    )(hidden_states, W, b)
